# Initial kernel scaffold; baseline (speedup 1.0000x reference)
import jax
import jax.numpy as jnp
from jax.experimental import pallas as pl


def kernel(query, key, value, proj_dir, sampled_set):
    raise NotImplementedError("write your pallas kernel here")


# speedup vs baseline: 11.2239x; 11.2239x over previous
import functools
import math

import jax
import jax.numpy as jnp
from jax.experimental import pallas as pl
from jax.experimental.pallas import tpu as pltpu

INPUT_DIM = 64
NUM_PROJS = 7
BLOCK_SIZE = 256
SAMPLE_SIZE = 256
N_SEQ = 8192
NUM_BLOCKS = N_SEQ // BLOCK_SIZE


def _attn_body(q_ref, kb_ref, vb_ref, ks_ref, vs_ref, samp_ref, out_ref):
    nb = pl.program_id(1)
    scale = INPUT_DIM ** (-0.5)
    qb = q_ref[0, 0]
    kb = kb_ref[0, 0]
    vb = vb_ref[0, 0]
    ks = ks_ref[0]
    vs = vs_ref[0]
    samp = samp_ref[0, 0]

    s1 = jax.lax.dot_general(qb, kb, (((1,), (1,)), ((), ())),
                             preferred_element_type=jnp.float32) * scale
    m1 = jnp.max(s1, axis=1, keepdims=True)
    p1 = jnp.exp(s1 - m1)
    l1 = jnp.sum(p1, axis=1, keepdims=True)
    a1 = jax.lax.dot_general(p1, vb, (((1,), (0,)), ((), ())),
                             preferred_element_type=jnp.float32)
    lse1 = m1 + jnp.log(l1)

    s2 = jax.lax.dot_general(qb, ks, (((1,), (1,)), ((), ())),
                             preferred_element_type=jnp.float32) * scale
    blk_of_samp = samp // BLOCK_SIZE
    neg = jnp.float32(jnp.finfo(jnp.float32).min)
    bias = jnp.where(blk_of_samp == nb, neg, jnp.float32(0.0))[None, :]
    s2 = s2 + bias
    m2 = jnp.max(s2, axis=1, keepdims=True)
    p2 = jnp.exp(s2 - m2)
    l2 = jnp.sum(p2, axis=1, keepdims=True)
    a2 = jax.lax.dot_general(p2, vs, (((1,), (0,)), ((), ())),
                             preferred_element_type=jnp.float32)
    lse2 = m2 + jnp.log(l2) + jnp.float32(math.log(N_SEQ / SAMPLE_SIZE))

    c = jax.nn.sigmoid(lse1 - lse2)
    out = c * (a1 / l1) + (1.0 - c) * (a2 / l2)
    out_ref[0, 0] = out


def _fused_attention(q_sorted, k_sorted, v_sorted, k_sub, v_sub, samp):
    BH, N, D = q_sorted.shape
    nb = NUM_BLOCKS
    qs4 = q_sorted.reshape(BH, nb, BLOCK_SIZE, D)
    ks4 = k_sorted.reshape(BH, nb, BLOCK_SIZE, D)
    vs4 = v_sorted.reshape(BH, nb, BLOCK_SIZE, D)
    grid = (BH, nb)
    blk = pl.BlockSpec((1, 1, BLOCK_SIZE, D), lambda i, j: (i, j, 0, 0))
    sub = pl.BlockSpec((1, SAMPLE_SIZE, D), lambda i, j: (i, 0, 0))
    sspec = pl.BlockSpec((1, 1, SAMPLE_SIZE), lambda i, j: (i, 0, 0))
    out = pl.pallas_call(
        _attn_body,
        grid=grid,
        in_specs=[blk, blk, blk, sub, sub, sspec],
        out_specs=blk,
        out_shape=jax.ShapeDtypeStruct((BH, nb, BLOCK_SIZE, D), jnp.float32),
    )(qs4, ks4, vs4, k_sub, v_sub, samp)
    return out.reshape(BH, N, D)


def _lsh_hash(mat, proj_dir):
    proj = jnp.einsum('bhnd,dr->bhnr', mat, proj_dir[:INPUT_DIM])
    bits = (proj > 0).astype(jnp.int32)
    enc = (2 ** jnp.arange(NUM_PROJS, dtype=jnp.int32))
    bin_ids = jnp.sum(bits * enc, axis=-1)
    return bin_ids ^ (bin_ids >> 1)


def kernel(query, key, value, proj_dir, sampled_set):
    B, H, N, D = query.shape
    BH = B * H
    q_hash = _lsh_hash(query, proj_dir)
    k_hash = _lsh_hash(key, proj_dir)
    q_sort = jnp.argsort(q_hash, axis=2)
    k_sort = jnp.argsort(k_hash, axis=2)

    q2 = query.reshape(BH, N, D)
    k2 = key.reshape(BH, N, D)
    v2 = value.reshape(BH, N, D)
    qs2 = q_sort.reshape(BH, N)
    ks2 = k_sort.reshape(BH, N)
    q_sorted = jnp.take_along_axis(q2, qs2[..., None], axis=1)
    k_sorted = jnp.take_along_axis(k2, ks2[..., None], axis=1)
    v_sorted = jnp.take_along_axis(v2, ks2[..., None], axis=1)

    samp2 = sampled_set.reshape(BH, SAMPLE_SIZE)
    k_sub = jnp.take_along_axis(k_sorted, samp2[..., None], axis=1)
    v_sub = jnp.take_along_axis(v_sorted, samp2[..., None], axis=1)

    merged = _fused_attention(q_sorted, k_sorted, v_sorted, k_sub, v_sub,
                              samp2.reshape(BH, 1, SAMPLE_SIZE))

    pos = jnp.argsort(qs2, axis=1)
    out = jnp.take_along_axis(merged, pos[..., None], axis=1)
    return out.reshape(B, H, N, D)
